# baseline (speedup 1.0000x reference)
import functools

import jax
import jax.numpy as jnp
import numpy as np
from jax.experimental import pallas as pl
from jax.experimental.pallas import tpu as pltpu

N = 10000
D = 128
B = 1024
NB = (N + B - 1) // B
LAST = N - (NB - 1) * B
NPAD = NB * B


def _build_schedule():
    cols = []
    for i in range(NB):
        js = list(range(i + 1, NB)) + list(range(0, i + 1))
        for idx, j in enumerate(js):
            first = 1 if idx == 0 else 0
            fin = 1 if j == i else 0
            p2 = 1 if j <= i else 0
            p2first = 1 if (j == 0 and i > 0) or (i == 0 and j == 0) else 0
            cols.append((i, j, 1, first, fin, p2, p2first, 1 if j == NB - 1 else 0))
    for i in range(NB):
        for j in range(i + 1, NB):
            cols.append((i, j, 0, 0, 0, 1, 0, 1 if j == NB - 1 else 0))
    return np.asarray(cols, dtype=np.int32).T.copy()


_SCHED = _build_schedule()
_T = _SCHED.shape[1]


def _fused_kernel(tbl_ref, a_ref, s0_ref, w1_ref, o_ref,
                  h_ref, g_ref, acc_ref):
    t = pl.program_id(0)
    ai = tbl_ref[0, t]
    aj = tbl_ref[1, t]

    a = a_ref[...]
    a = jax.lax.cond(
        tbl_ref[7, t] == 1,
        lambda v: jnp.where(
            jax.lax.broadcasted_iota(jnp.int32, v.shape, 1) < LAST, v, 0.0),
        lambda v: v,
        a)
    ab = a.astype(jnp.bfloat16)

    @pl.when(tbl_ref[2, t] == 1)
    def _layer1():
        contrib = jnp.dot(ab, s0_ref[pl.ds(aj * B, B), :],
                          preferred_element_type=jnp.float32)

        @pl.when(tbl_ref[3, t] == 1)
        def _():
            h_ref[...] = contrib

        @pl.when(tbl_ref[3, t] == 0)
        def _():
            h_ref[...] += contrib

    @pl.when(tbl_ref[4, t] == 1)
    def _finalize_g():
        hh = jnp.maximum(h_ref[...], 0.0)
        limit = jnp.where(ai == NB - 1, LAST, B)
        hh = jnp.where(
            jax.lax.broadcasted_iota(jnp.int32, hh.shape, 0) < limit, hh, 0.0)
        gblk = jnp.dot(hh, w1_ref[...], preferred_element_type=jnp.float32)
        g_ref[pl.ds(ai * B, B), :] = gblk.astype(jnp.bfloat16)

    @pl.when(tbl_ref[5, t] == 1)
    def _layer2():
        contrib = jnp.dot(ab, g_ref[pl.ds(aj * B, B), :],
                          preferred_element_type=jnp.float32)

        @pl.when(tbl_ref[6, t] == 1)
        def _():
            acc_ref[pl.ds(ai * B, B), :] = contrib

        @pl.when(tbl_ref[6, t] == 0)
        def _():
            acc_ref[pl.ds(ai * B, B), :] += contrib

    @pl.when(t == _T - 1)
    def _writeout():
        o_ref[...] = acc_ref[:N, :]


def _mm_small_kernel(x_ref, w_ref, o_ref):
    o_ref[...] = jnp.dot(x_ref[...], w_ref[...],
                         preferred_element_type=jnp.float32)


def _small_matmul(x, w):
    rb = 1000
    return pl.pallas_call(
        _mm_small_kernel,
        grid=(N // rb,),
        in_specs=[pl.BlockSpec((rb, D), lambda i: (i, 0)),
                  pl.BlockSpec((D, D), lambda i: (0, 0))],
        out_specs=pl.BlockSpec((rb, D), lambda i: (i, 0)),
        out_shape=jax.ShapeDtypeStruct((N, D), jnp.float32),
    )(x, w)


def kernel(x, adjacency, W0, W1):
    s0 = _small_matmul(x, W0)
    s0p = jnp.zeros((NPAD, D), jnp.bfloat16).at[:N].set(s0.astype(jnp.bfloat16))
    tbl = jnp.asarray(_SCHED)
    grid_spec = pltpu.PrefetchScalarGridSpec(
        num_scalar_prefetch=1,
        grid=(_T,),
        in_specs=[
            pl.BlockSpec((B, B), lambda t, tbl: (tbl[0, t], tbl[1, t])),
            pl.BlockSpec((NPAD, D), lambda t, tbl: (0, 0)),
            pl.BlockSpec((D, D), lambda t, tbl: (0, 0)),
        ],
        out_specs=pl.BlockSpec((N, D), lambda t, tbl: (0, 0)),
        scratch_shapes=[
            pltpu.VMEM((B, D), jnp.float32),
            pltpu.VMEM((NPAD, D), jnp.bfloat16),
            pltpu.VMEM((NPAD, D), jnp.float32),
        ],
    )
    return pl.pallas_call(
        _fused_kernel,
        grid_spec=grid_spec,
        out_shape=jax.ShapeDtypeStruct((N, D), jnp.float32),
        compiler_params=pltpu.CompilerParams(
            dimension_semantics=("arbitrary",)),
    )(tbl, adjacency, s0p, W1)


# speedup vs baseline: 1.6474x; 1.6474x over previous
import jax
import jax.numpy as jnp
import numpy as np
from jax.experimental import pallas as pl
from jax.experimental.pallas import tpu as pltpu

N = 10000
D = 128
B = 1024
NB = (N + B - 1) // B
LAST = N - (NB - 1) * B
NPAD = NB * B


def _build_schedule():
    cols = []
    for i in range(NB):
        js = list(range(i + 1, NB)) + list(range(0, i + 1))
        for idx, j in enumerate(js):
            first = 1 if idx == 0 else 0
            fin = 1 if j == i else 0
            p2 = 1 if j < i else 0
            p2init = 1 if j == 0 else 0
            cols.append((i, j, 1, first, fin, p2, p2init, 1 if j == NB - 1 else 0))
    for i in range(NB):
        for j in range(i + 1, NB):
            cols.append((i, j, 0, 0, 0, 1, 0, 1 if j == NB - 1 else 0))
    return np.asarray(cols, dtype=np.int32).T.copy()


_SCHED = _build_schedule()
_T = _SCHED.shape[1]


def _fused_kernel(tbl_ref, a_ref, s0_ref, w1_ref, o_ref,
                  h_ref, sg_ref, acc_ref):
    t = pl.program_id(0)
    ai = tbl_ref[0, t]
    aj = tbl_ref[1, t]

    @pl.when(t == 0)
    def _load_s0():
        sg_ref[:, :D] = s0_ref[...]

    @pl.when(tbl_ref[7, t] == 1)
    def _zero_pad_cols():
        a_ref[:, LAST:] = jnp.zeros((B, B - LAST), jnp.float32)

    both = jnp.dot(a_ref[...], sg_ref[pl.ds(aj * B, B), :],
                   preferred_element_type=jnp.float32)

    @pl.when(tbl_ref[2, t] == 1)
    def _layer1():
        @pl.when(tbl_ref[3, t] == 1)
        def _():
            h_ref[...] = both[:, :D]

        @pl.when(tbl_ref[3, t] == 0)
        def _():
            h_ref[...] += both[:, :D]

    @pl.when(tbl_ref[4, t] == 1)
    def _finalize_g():
        hh = jnp.maximum(h_ref[...], 0.0)
        limit = jnp.where(ai == NB - 1, LAST, B)
        hh = jnp.where(
            jax.lax.broadcasted_iota(jnp.int32, hh.shape, 0) < limit, hh, 0.0)
        gblk = jnp.dot(hh, w1_ref[...], preferred_element_type=jnp.float32)
        sg_ref[pl.ds(ai * B, B), D:] = gblk
        diag = jnp.dot(a_ref[...], gblk, preferred_element_type=jnp.float32)

        @pl.when(tbl_ref[6, t] == 1)
        def _():
            acc_ref[pl.ds(ai * B, B), :] = diag

        @pl.when(tbl_ref[6, t] == 0)
        def _():
            acc_ref[pl.ds(ai * B, B), :] += diag

    @pl.when(tbl_ref[5, t] == 1)
    def _layer2():
        @pl.when(tbl_ref[6, t] == 1)
        def _():
            acc_ref[pl.ds(ai * B, B), :] = both[:, D:]

        @pl.when(tbl_ref[6, t] == 0)
        def _():
            acc_ref[pl.ds(ai * B, B), :] += both[:, D:]

    @pl.when(t == _T - 1)
    def _writeout():
        o_ref[...] = acc_ref[:N, :]


def _mm_small_kernel(x_ref, w_ref, o_ref):
    o_ref[...] = jnp.dot(x_ref[...], w_ref[...],
                         preferred_element_type=jnp.float32)


def _small_matmul(x, w):
    rb = 1000
    return pl.pallas_call(
        _mm_small_kernel,
        grid=(N // rb,),
        in_specs=[pl.BlockSpec((rb, D), lambda i: (i, 0)),
                  pl.BlockSpec((D, D), lambda i: (0, 0))],
        out_specs=pl.BlockSpec((rb, D), lambda i: (i, 0)),
        out_shape=jax.ShapeDtypeStruct((N, D), jnp.float32),
    )(x, w)


def kernel(x, adjacency, W0, W1):
    s0 = _small_matmul(x, W0)
    s0p = jnp.zeros((NPAD, D), jnp.float32).at[:N].set(s0)
    tbl = jnp.asarray(_SCHED)
    grid_spec = pltpu.PrefetchScalarGridSpec(
        num_scalar_prefetch=1,
        grid=(_T,),
        in_specs=[
            pl.BlockSpec((B, B), lambda t, tbl: (tbl[0, t], tbl[1, t])),
            pl.BlockSpec((NPAD, D), lambda t, tbl: (0, 0)),
            pl.BlockSpec((D, D), lambda t, tbl: (0, 0)),
        ],
        out_specs=pl.BlockSpec((N, D), lambda t, tbl: (0, 0)),
        scratch_shapes=[
            pltpu.VMEM((B, D), jnp.float32),
            pltpu.VMEM((NPAD, 2 * D), jnp.float32),
            pltpu.VMEM((NPAD, D), jnp.float32),
        ],
    )
    return pl.pallas_call(
        _fused_kernel,
        grid_spec=grid_spec,
        out_shape=jax.ShapeDtypeStruct((N, D), jnp.float32),
        compiler_params=pltpu.CompilerParams(
            dimension_semantics=("arbitrary",)),
    )(tbl, adjacency, s0p, W1)


# speedup vs baseline: 1.9981x; 1.2129x over previous
import jax
import jax.numpy as jnp
import numpy as np
from jax.experimental import pallas as pl
from jax.experimental.pallas import tpu as pltpu

N = 10000
D = 128
BR = 1024
BC = 2048
NBR = (N + BR - 1) // BR
NBC = (N + BC - 1) // BC
LASTR = N - (NBR - 1) * BR
LASTC = N - (NBC - 1) * BC
NPAD = NBR * BR


def _build_schedule():
    cols = []
    touched = [False] * NBR
    for i in range(NBR):
        ready = [j for j in range(NBC) if BC * (j + 1) <= BR * i]
        jd = (i - 1) // 2 if i % 2 == 1 else None
        others = [j for j in range(NBC) if j not in ready and j != jd]
        order = others + ready + ([jd] if jd is not None else [])
        for idx, j in enumerate(order):
            first = 1 if idx == 0 else 0
            fin = 1 if idx == len(order) - 1 else 0
            p2 = 1 if j in ready else 0
            p2init = 1 if (p2 and not touched[i]) else 0
            if p2:
                touched[i] = True
            fe = 1 if (fin and jd is not None) else 0
            feinit = 1 if (fe and not touched[i]) else 0
            if fe:
                touched[i] = True
            cols.append((i, j, 1, first, fin, p2, p2init,
                         1 if j == NBC - 1 else 0, fe, feinit))
    for i in range(NBR):
        ready = [j for j in range(NBC) if BC * (j + 1) <= BR * i]
        jd = (i - 1) // 2 if i % 2 == 1 else None
        for j in range(NBC):
            if j in ready or j == jd:
                continue
            p2init = 1 if not touched[i] else 0
            touched[i] = True
            cols.append((i, j, 0, 0, 0, 1, p2init,
                         1 if j == NBC - 1 else 0, 0, 0))
    return np.asarray(cols, dtype=np.int32).T.copy()


_SCHED = _build_schedule()
_T = _SCHED.shape[1]


def _fused_kernel(tbl_ref, a_ref, s0_ref, w1_ref, o_ref,
                  h_ref, sg_ref, acc_ref):
    t = pl.program_id(0)
    ai = tbl_ref[0, t]
    aj = tbl_ref[1, t]

    @pl.when(t == 0)
    def _load_s0():
        sg_ref[:, :D] = s0_ref[...]

    @pl.when(tbl_ref[7, t] == 1)
    def _zero_pad_cols():
        a_ref[:, LASTC:] = jnp.zeros((BR, BC - LASTC), jnp.float32)

    both = jnp.dot(a_ref[...], sg_ref[pl.ds(aj * BC, BC), :],
                   preferred_element_type=jnp.float32)

    @pl.when(tbl_ref[2, t] == 1)
    def _layer1():
        @pl.when(tbl_ref[3, t] == 1)
        def _():
            h_ref[...] = both[:, :D]

        @pl.when(tbl_ref[3, t] == 0)
        def _():
            h_ref[...] += both[:, :D]

    @pl.when(tbl_ref[4, t] == 1)
    def _finalize_g():
        hh = jnp.maximum(h_ref[...], 0.0)
        limit = jnp.where(ai == NBR - 1, LASTR, BR)
        hh = jnp.where(
            jax.lax.broadcasted_iota(jnp.int32, hh.shape, 0) < limit, hh, 0.0)
        sg_ref[pl.ds(ai * BR, BR), D:] = jnp.dot(
            hh, w1_ref[...], preferred_element_type=jnp.float32)

    @pl.when(tbl_ref[8, t] == 1)
    def _diag_extra():
        extra = jnp.dot(a_ref[...], sg_ref[pl.ds(aj * BC, BC), D:],
                        preferred_element_type=jnp.float32)

        @pl.when(tbl_ref[9, t] == 1)
        def _():
            acc_ref[pl.ds(ai * BR, BR), :] = extra

        @pl.when(tbl_ref[9, t] == 0)
        def _():
            acc_ref[pl.ds(ai * BR, BR), :] += extra

    @pl.when(tbl_ref[5, t] == 1)
    def _layer2():
        @pl.when(tbl_ref[6, t] == 1)
        def _():
            acc_ref[pl.ds(ai * BR, BR), :] = both[:, D:]

        @pl.when(tbl_ref[6, t] == 0)
        def _():
            acc_ref[pl.ds(ai * BR, BR), :] += both[:, D:]

    @pl.when(t == _T - 1)
    def _writeout():
        o_ref[...] = acc_ref[:N, :]


def _mm_small_kernel(x_ref, w_ref, o_ref):
    o_ref[...] = jnp.dot(x_ref[...], w_ref[...],
                         preferred_element_type=jnp.float32)


def _small_matmul(x, w):
    rb = 1000
    return pl.pallas_call(
        _mm_small_kernel,
        grid=(N // rb,),
        in_specs=[pl.BlockSpec((rb, D), lambda i: (i, 0)),
                  pl.BlockSpec((D, D), lambda i: (0, 0))],
        out_specs=pl.BlockSpec((rb, D), lambda i: (i, 0)),
        out_shape=jax.ShapeDtypeStruct((N, D), jnp.float32),
    )(x, w)


def kernel(x, adjacency, W0, W1):
    s0 = _small_matmul(x, W0)
    s0p = jnp.zeros((NPAD, D), jnp.float32).at[:N].set(s0)
    tbl = jnp.asarray(_SCHED)
    grid_spec = pltpu.PrefetchScalarGridSpec(
        num_scalar_prefetch=1,
        grid=(_T,),
        in_specs=[
            pl.BlockSpec((BR, BC), lambda t, tbl: (tbl[0, t], tbl[1, t])),
            pl.BlockSpec((NPAD, D), lambda t, tbl: (0, 0)),
            pl.BlockSpec((D, D), lambda t, tbl: (0, 0)),
        ],
        out_specs=pl.BlockSpec((N, D), lambda t, tbl: (0, 0)),
        scratch_shapes=[
            pltpu.VMEM((BR, D), jnp.float32),
            pltpu.VMEM((NPAD, 2 * D), jnp.float32),
            pltpu.VMEM((NPAD, D), jnp.float32),
        ],
    )
    return pl.pallas_call(
        _fused_kernel,
        grid_spec=grid_spec,
        out_shape=jax.ShapeDtypeStruct((N, D), jnp.float32),
        compiler_params=pltpu.CompilerParams(
            dimension_semantics=("arbitrary",)),
    )(tbl, adjacency, s0p, W1)


# speedup vs baseline: 2.0985x; 1.0502x over previous
import jax
import jax.numpy as jnp
import numpy as np
from jax.experimental import pallas as pl
from jax.experimental.pallas import tpu as pltpu

N = 10000
D = 128
B = 2048
NB = (N + B - 1) // B
LAST = N - (NB - 1) * B
NPAD = NB * B


def _build_schedule():
    cols = []
    for i in range(NB):
        js = list(range(i + 1, NB)) + list(range(0, i)) + [i]
        for idx, j in enumerate(js):
            first = 1 if idx == 0 else 0
            fin = 1 if j == i else 0
            p2 = 1 if j < i else 0
            p2init = 1 if (p2 and j == 0) else 0
            fe = fin
            feinit = 1 if (fe and i == 0) else 0
            owrite = 1 if (fin and i == NB - 1) else 0
            cols.append((i, j, 1, first, fin, p2, p2init,
                         1 if j == NB - 1 else 0, fe, feinit, NB - 1, owrite))
    for i in range(NB):
        for j in range(i + 1, NB):
            owrite = 1 if j == NB - 1 else 0
            cols.append((i, j, 0, 0, 0, 1, 0,
                         1 if j == NB - 1 else 0, 0, 0, i, owrite))
    return np.asarray(cols, dtype=np.int32).T.copy()


_SCHED = _build_schedule()
_T = _SCHED.shape[1]


def _fused_kernel(tbl_ref, a_ref, sg_ref, w1_ref, o_ref,
                  h_ref, acc_ref):
    t = pl.program_id(0)
    ai = tbl_ref[0, t]
    aj = tbl_ref[1, t]

    @pl.when(tbl_ref[7, t] == 1)
    def _zero_pad_cols():
        a_ref[:, LAST:] = jnp.zeros((B, B - LAST), jnp.float32)

    both = jnp.dot(a_ref[...], sg_ref[pl.ds(aj * B, B), :],
                   preferred_element_type=jnp.float32)

    @pl.when(tbl_ref[2, t] == 1)
    def _layer1():
        @pl.when(tbl_ref[3, t] == 1)
        def _():
            h_ref[...] = both[:, :D]

        @pl.when(tbl_ref[3, t] == 0)
        def _():
            h_ref[...] += both[:, :D]

    @pl.when(tbl_ref[4, t] == 1)
    def _finalize_g():
        hh = jnp.maximum(h_ref[...], 0.0)
        limit = jnp.where(ai == NB - 1, LAST, B)
        hh = jnp.where(
            jax.lax.broadcasted_iota(jnp.int32, hh.shape, 0) < limit, hh, 0.0)
        sg_ref[pl.ds(ai * B, B), D:] = jnp.dot(
            hh, w1_ref[...], preferred_element_type=jnp.float32)

    @pl.when(tbl_ref[8, t] == 1)
    def _diag_extra():
        extra = jnp.dot(a_ref[...], sg_ref[pl.ds(aj * B, B), D:],
                        preferred_element_type=jnp.float32)

        @pl.when(tbl_ref[9, t] == 1)
        def _():
            acc_ref[pl.ds(ai * B, B), :] = extra

        @pl.when(tbl_ref[9, t] == 0)
        def _():
            acc_ref[pl.ds(ai * B, B), :] += extra

    @pl.when(tbl_ref[5, t] == 1)
    def _layer2():
        @pl.when(tbl_ref[6, t] == 1)
        def _():
            acc_ref[pl.ds(ai * B, B), :] = both[:, D:]

        @pl.when(tbl_ref[6, t] == 0)
        def _():
            acc_ref[pl.ds(ai * B, B), :] += both[:, D:]

    @pl.when(tbl_ref[11, t] == 1)
    def _writeout():
        o_ref[...] = acc_ref[pl.ds(ai * B, B), :]


def _mm_small_kernel(x_ref, w_ref, o_ref):
    o_ref[...] = jnp.dot(x_ref[...], w_ref[...],
                         preferred_element_type=jnp.float32)


def _small_matmul(x, w):
    rb = 1000
    return pl.pallas_call(
        _mm_small_kernel,
        grid=(N // rb,),
        in_specs=[pl.BlockSpec((rb, D), lambda i: (i, 0)),
                  pl.BlockSpec((D, D), lambda i: (0, 0))],
        out_specs=pl.BlockSpec((rb, D), lambda i: (i, 0)),
        out_shape=jax.ShapeDtypeStruct((N, D), jnp.float32),
    )(x, w)


def kernel(x, adjacency, W0, W1):
    s0 = _small_matmul(x, W0)
    sgp = jnp.zeros((NPAD, 2 * D), jnp.float32).at[:N, :D].set(s0)
    tbl = jnp.asarray(_SCHED)
    grid_spec = pltpu.PrefetchScalarGridSpec(
        num_scalar_prefetch=1,
        grid=(_T,),
        in_specs=[
            pl.BlockSpec((B, B), lambda t, tbl: (tbl[0, t], tbl[1, t])),
            pl.BlockSpec((NPAD, 2 * D), lambda t, tbl: (0, 0)),
            pl.BlockSpec((D, D), lambda t, tbl: (0, 0)),
        ],
        out_specs=pl.BlockSpec((B, D), lambda t, tbl: (tbl[10, t], 0)),
        scratch_shapes=[
            pltpu.VMEM((B, D), jnp.float32),
            pltpu.VMEM((NPAD, D), jnp.float32),
        ],
    )
    return pl.pallas_call(
        _fused_kernel,
        grid_spec=grid_spec,
        out_shape=jax.ShapeDtypeStruct((N, D), jnp.float32),
        compiler_params=pltpu.CompilerParams(
            dimension_semantics=("arbitrary",)),
    )(tbl, adjacency, sgp, W1)


# speedup vs baseline: 2.1569x; 1.0278x over previous
import jax
import jax.numpy as jnp
import numpy as np
from jax.experimental import pallas as pl
from jax.experimental.pallas import tpu as pltpu

N = 10000
D = 128
B = 2048
NB = (N + B - 1) // B
LAST = N - (NB - 1) * B
NPAD = NB * B


def _build_schedule():
    cols = []
    for i in range(NB):
        js = list(range(i + 1, NB)) + list(range(0, i)) + [i]
        for idx, j in enumerate(js):
            first = 1 if idx == 0 else 0
            fin = 1 if j == i else 0
            p2 = 1 if j < i else 0
            p2init = 1 if (p2 and j == 0) else 0
            fe = fin
            feinit = 1 if (fe and i == 0) else 0
            owrite = 1 if (fin and i == NB - 1) else 0
            cols.append((i, j, 1, first, fin, p2, p2init,
                         1 if j == NB - 1 else 0, fe, feinit, NB - 1, owrite))
    for i in range(NB):
        for j in range(i + 1, NB):
            owrite = 1 if j == NB - 1 else 0
            cols.append((i, j, 0, 0, 0, 1, 0,
                         1 if j == NB - 1 else 0, 0, 0, i, owrite))
    return np.asarray(cols, dtype=np.int32).T.copy()


_SCHED = _build_schedule()
_T = _SCHED.shape[1]


def _fused_kernel(tbl_ref, a_ref, sg_ref, w1_ref, o_ref,
                  h_ref, acc_ref):
    t = pl.program_id(0)
    ai = tbl_ref[0, t]
    aj = tbl_ref[1, t]

    @pl.when(tbl_ref[7, t] == 1)
    def _zero_pad_cols():
        a_ref[:, LAST:] = jnp.zeros((B, B - LAST), jnp.float32)

    both = jnp.dot(a_ref[...], sg_ref[pl.ds(aj * B, B), :],
                   preferred_element_type=jnp.float32)

    @pl.when(tbl_ref[2, t] == 1)
    def _layer1():
        @pl.when(tbl_ref[3, t] == 1)
        def _():
            h_ref[...] = both[:, :D]

        @pl.when(tbl_ref[3, t] == 0)
        def _():
            h_ref[...] += both[:, :D]

    @pl.when(tbl_ref[4, t] == 1)
    def _finalize_g():
        hh = jnp.maximum(h_ref[...], 0.0)
        limit = jnp.where(ai == NB - 1, LAST, B)
        hh = jnp.where(
            jax.lax.broadcasted_iota(jnp.int32, hh.shape, 0) < limit, hh, 0.0)
        sg_ref[pl.ds(ai * B, B), D:] = jnp.dot(
            hh, w1_ref[...], preferred_element_type=jnp.float32)

    @pl.when(tbl_ref[8, t] == 1)
    def _diag_extra():
        extra = jnp.dot(a_ref[...], sg_ref[pl.ds(aj * B, B), D:],
                        preferred_element_type=jnp.float32)

        @pl.when(tbl_ref[9, t] == 1)
        def _():
            acc_ref[pl.ds(ai * B, B), :] = extra

        @pl.when(tbl_ref[9, t] == 0)
        def _():
            acc_ref[pl.ds(ai * B, B), :] += extra

    @pl.when(tbl_ref[5, t] == 1)
    def _layer2():
        @pl.when(tbl_ref[6, t] == 1)
        def _():
            acc_ref[pl.ds(ai * B, B), :] = both[:, D:]

        @pl.when(tbl_ref[6, t] == 0)
        def _():
            acc_ref[pl.ds(ai * B, B), :] += both[:, D:]

    @pl.when(tbl_ref[11, t] == 1)
    def _writeout():
        o_ref[...] = acc_ref[pl.ds(ai * B, B), :]


_PB = 1024


def _prep_kernel(x_ref, w_ref, o_ref):
    i = pl.program_id(0)
    s = jnp.dot(x_ref[...], w_ref[...], preferred_element_type=jnp.float32)
    limit = N - i * _PB
    s = jnp.where(
        jax.lax.broadcasted_iota(jnp.int32, s.shape, 0) < limit, s, 0.0)
    o_ref[:, :D] = s
    o_ref[:, D:] = jnp.zeros_like(s)


def _prep_sg(x, w0):
    return pl.pallas_call(
        _prep_kernel,
        grid=(NPAD // _PB,),
        in_specs=[pl.BlockSpec((_PB, D), lambda i: (i, 0)),
                  pl.BlockSpec((D, D), lambda i: (0, 0))],
        out_specs=pl.BlockSpec((_PB, 2 * D), lambda i: (i, 0)),
        out_shape=jax.ShapeDtypeStruct((NPAD, 2 * D), jnp.float32),
    )(x, w0)


def kernel(x, adjacency, W0, W1):
    sgp = _prep_sg(x, W0)
    tbl = jnp.asarray(_SCHED)
    grid_spec = pltpu.PrefetchScalarGridSpec(
        num_scalar_prefetch=1,
        grid=(_T,),
        in_specs=[
            pl.BlockSpec((B, B), lambda t, tbl: (tbl[0, t], tbl[1, t])),
            pl.BlockSpec((NPAD, 2 * D), lambda t, tbl: (0, 0)),
            pl.BlockSpec((D, D), lambda t, tbl: (0, 0)),
        ],
        out_specs=pl.BlockSpec((B, D), lambda t, tbl: (tbl[10, t], 0)),
        scratch_shapes=[
            pltpu.VMEM((B, D), jnp.float32),
            pltpu.VMEM((NPAD, D), jnp.float32),
        ],
    )
    return pl.pallas_call(
        _fused_kernel,
        grid_spec=grid_spec,
        out_shape=jax.ShapeDtypeStruct((N, D), jnp.float32),
        compiler_params=pltpu.CompilerParams(
            dimension_semantics=("arbitrary",)),
    )(tbl, adjacency, sgp, W1)


# speedup vs baseline: 2.2322x; 1.0349x over previous
import jax
import jax.numpy as jnp
import numpy as np
from jax.experimental import pallas as pl
from jax.experimental.pallas import tpu as pltpu

N = 10000
D = 128
B = 2048
NB = (N + B - 1) // B
LAST = N - (NB - 1) * B
NPAD = NB * B


def _build_schedule():
    cols = []
    t = 0
    for i in range(NB):
        js = list(range(i + 1, NB)) + list(range(0, i)) + [i]
        for idx, j in enumerate(js):
            first = 1 if idx == 0 else 0
            fin = 1 if j == i else 0
            p2 = 1 if j < i else 0
            p2init = 1 if (p2 and j == 0) else 0
            fe = fin
            feinit = 1 if (fe and i == 0) else 0
            owrite = 1 if (fin and i == NB - 1) else 0
            xi = j if t < NB else 0
            sinit = 1 if t < NB else 0
            cols.append((i, j, 1, first, fin, p2, p2init,
                         1 if j == NB - 1 else 0, fe, feinit, NB - 1, owrite,
                         xi, sinit))
            t += 1
    for i in range(NB):
        for j in range(i + 1, NB):
            owrite = 1 if j == NB - 1 else 0
            cols.append((i, j, 0, 0, 0, 1, 0,
                         1 if j == NB - 1 else 0, 0, 0, i, owrite, 0, 0))
    return np.asarray(cols, dtype=np.int32).T.copy()


_SCHED = _build_schedule()
_T = _SCHED.shape[1]


def _fused_kernel(tbl_ref, a_ref, x_ref, w0_ref, w1_ref, o_ref,
                  sg_ref, h_ref):
    t = pl.program_id(0)
    ai = tbl_ref[0, t]
    aj = tbl_ref[1, t]

    @pl.when(tbl_ref[13, t] == 1)
    def _init_sg_block():
        xi = tbl_ref[12, t]
        s = jnp.dot(x_ref[...], w0_ref[...],
                    preferred_element_type=jnp.float32)
        limit = jnp.where(xi == NB - 1, LAST, B)
        s = jnp.where(
            jax.lax.broadcasted_iota(jnp.int32, s.shape, 0) < limit, s, 0.0)
        sg_ref[pl.ds(xi * B, B), :D] = s
        sg_ref[pl.ds(xi * B, B), D:] = jnp.zeros((B, D), jnp.float32)

    @pl.when(tbl_ref[7, t] == 1)
    def _zero_pad_cols():
        a_ref[:, LAST:] = jnp.zeros((B, B - LAST), jnp.float32)

    both = jnp.dot(a_ref[...], sg_ref[pl.ds(aj * B, B), :],
                   preferred_element_type=jnp.float32)

    @pl.when(tbl_ref[2, t] == 1)
    def _layer1():
        @pl.when(tbl_ref[3, t] == 1)
        def _():
            h_ref[...] = both[:, :D]

        @pl.when(tbl_ref[3, t] == 0)
        def _():
            h_ref[...] += both[:, :D]

    @pl.when(tbl_ref[4, t] == 1)
    def _finalize_g():
        hh = jnp.maximum(h_ref[...], 0.0)
        limit = jnp.where(ai == NB - 1, LAST, B)
        hh = jnp.where(
            jax.lax.broadcasted_iota(jnp.int32, hh.shape, 0) < limit, hh, 0.0)
        sg_ref[pl.ds(ai * B, B), D:] = jnp.dot(
            hh, w1_ref[...], preferred_element_type=jnp.float32)

    @pl.when(tbl_ref[8, t] == 1)
    def _diag_extra():
        extra = jnp.dot(a_ref[...], sg_ref[pl.ds(aj * B, B), D:],
                        preferred_element_type=jnp.float32)

        @pl.when(tbl_ref[9, t] == 1)
        def _():
            o_ref[pl.ds(ai * B, B), :] = extra

        @pl.when(tbl_ref[9, t] == 0)
        def _():
            o_ref[pl.ds(ai * B, B), :] += extra

    @pl.when(tbl_ref[5, t] == 1)
    def _layer2():
        @pl.when(tbl_ref[6, t] == 1)
        def _():
            o_ref[pl.ds(ai * B, B), :] = both[:, D:]

        @pl.when(tbl_ref[6, t] == 0)
        def _():
            o_ref[pl.ds(ai * B, B), :] += both[:, D:]


def kernel(x, adjacency, W0, W1):
    tbl = jnp.asarray(_SCHED)
    grid_spec = pltpu.PrefetchScalarGridSpec(
        num_scalar_prefetch=1,
        grid=(_T,),
        in_specs=[
            pl.BlockSpec((B, B), lambda t, tbl: (tbl[0, t], tbl[1, t])),
            pl.BlockSpec((B, D), lambda t, tbl: (tbl[12, t], 0)),
            pl.BlockSpec((D, D), lambda t, tbl: (0, 0)),
            pl.BlockSpec((D, D), lambda t, tbl: (0, 0)),
        ],
        out_specs=pl.BlockSpec((NPAD, D), lambda t, tbl: (0, 0)),
        scratch_shapes=[
            pltpu.VMEM((NPAD, 2 * D), jnp.float32),
            pltpu.VMEM((B, D), jnp.float32),
        ],
    )
    out = pl.pallas_call(
        _fused_kernel,
        grid_spec=grid_spec,
        out_shape=jax.ShapeDtypeStruct((NPAD, D), jnp.float32),
        compiler_params=pltpu.CompilerParams(
            dimension_semantics=("arbitrary",)),
    )(tbl, adjacency, x, W0, W1)
    return out[:N]
